# baseline (speedup 1.0000x reference)
import jax
import jax.numpy as jnp
from jax import lax
from jax.experimental import pallas as pl
from jax.experimental.pallas import tpu as pltpu
from jax.experimental.pallas import tpu_sc as plsc

NUM_CORES = 2
NUM_SUBCORES = 16
NW = NUM_CORES * NUM_SUBCORES
BATCH = 16384
PER_W = BATCH // NW
NCHUNK = PER_W // 128
D = 64
GROUPS = PER_W // 16


def _body(users_r, pos_r, neg_r, uemb_r, iemb_r, ibias_r,
          outp_hbm, outn_hbm,
          idx_u, idx_p, idx_n, rows_u, rows_p, rows_n,
          bias_p, bias_n, outp_v, outn_v, sem):
    c = lax.axis_index("c")
    s = lax.axis_index("s")
    wid = s * NUM_CORES + c

    pltpu.sync_copy(users_r.at[wid], idx_u)
    pltpu.sync_copy(pos_r.at[wid], idx_p)
    pltpu.sync_copy(neg_r.at[wid], idx_n)

    copies = []
    for j in range(NCHUNK):
        dst = pl.ds(j * 128, 128)
        copies.append(pltpu.async_copy(uemb_r.at[idx_u.at[j]], rows_u.at[dst], sem))
        copies.append(pltpu.async_copy(iemb_r.at[idx_p.at[j]], rows_p.at[dst], sem))
        copies.append(pltpu.async_copy(iemb_r.at[idx_n.at[j]], rows_n.at[dst], sem))
        copies.append(pltpu.async_copy(ibias_r.at[idx_p.at[j]], bias_p.at[dst], sem))
        copies.append(pltpu.async_copy(ibias_r.at[idx_n.at[j]], bias_n.at[dst], sem))
    for cp in copies:
        cp.wait()

    iota16 = lax.iota(jnp.int32, 16)

    def group(g, carry):
        rows16 = g * 16 + iota16
        accp = bias_p[pl.ds(g * 16, 16)]
        accn = bias_n[pl.ds(g * 16, 16)]
        for d in range(D):
            cols = jnp.full((16,), d, jnp.int32)
            ul = plsc.load_gather(rows_u, [rows16, cols])
            pv = plsc.load_gather(rows_p, [rows16, cols])
            nv = plsc.load_gather(rows_n, [rows16, cols])
            accp = accp + ul * pv
            accn = accn + ul * nv
        outp_v[pl.ds(g * 16, 16)] = accp
        outn_v[pl.ds(g * 16, 16)] = accn
        return carry

    lax.fori_loop(0, GROUPS, group, 0)

    pltpu.sync_copy(outp_v, outp_hbm.at[pl.ds(wid * PER_W, PER_W)])
    pltpu.sync_copy(outn_v, outn_hbm.at[pl.ds(wid * PER_W, PER_W)])


_mesh = plsc.VectorSubcoreMesh(core_axis_name="c", subcore_axis_name="s",
                               num_cores=NUM_CORES, num_subcores=NUM_SUBCORES)

_sc_call = pl.kernel(
    _body,
    out_type=(jax.ShapeDtypeStruct((BATCH,), jnp.float32),
              jax.ShapeDtypeStruct((BATCH,), jnp.float32)),
    mesh=_mesh,
    scratch_types=[
        pltpu.VMEM((NCHUNK, 128), jnp.int32),
        pltpu.VMEM((NCHUNK, 128), jnp.int32),
        pltpu.VMEM((NCHUNK, 128), jnp.int32),
        pltpu.VMEM((PER_W, D), jnp.float32),
        pltpu.VMEM((PER_W, D), jnp.float32),
        pltpu.VMEM((PER_W, D), jnp.float32),
        pltpu.VMEM((PER_W,), jnp.float32),
        pltpu.VMEM((PER_W,), jnp.float32),
        pltpu.VMEM((PER_W,), jnp.float32),
        pltpu.VMEM((PER_W,), jnp.float32),
        pltpu.SemaphoreType.DMA,
    ],
    compiler_params=pltpu.CompilerParams(needs_layout_passes=False,
                                         use_tc_tiling_on_sc=False),
    name="bprmf_sc",
)


def kernel(users, pos_items, neg_items, user_emb, item_emb, item_bias):
    users3 = users.astype(jnp.int32).reshape(NW, NCHUNK, 128)
    pos3 = pos_items.astype(jnp.int32).reshape(NW, NCHUNK, 128)
    neg3 = neg_items.astype(jnp.int32).reshape(NW, NCHUNK, 128)
    bias_flat = item_bias.reshape(-1)
    return _sc_call(users3, pos3, neg3, user_emb, item_emb, bias_flat)


# speedup vs baseline: 1.1875x; 1.1875x over previous
import jax
import jax.numpy as jnp
from jax import lax
from jax.experimental import pallas as pl
from jax.experimental.pallas import tpu as pltpu
from jax.experimental.pallas import tpu_sc as plsc

NUM_CORES = 2
NUM_SUBCORES = 16
NW = NUM_CORES * NUM_SUBCORES
BATCH = 16384
PER_W = BATCH // NW
NROW = 8
NCOL = PER_W // NROW
D = 64
NCHUNK = 4
CHUNK = PER_W // NCHUNK
CGROUP = CHUNK // 16


def _body(users_r, pos_r, neg_r, uemb_r, iemb_r, ibias_r,
          outp_hbm, outn_hbm,
          idx_u, idx_p, idx_n,
          rows_u, rows_p, rows_n,
          bval_p, bval_n, outp_v, outn_v, sem):
    c = lax.axis_index("c")
    s = lax.axis_index("s")
    wid = s * NUM_CORES + c

    pltpu.sync_copy(users_r.at[wid], idx_u)
    pltpu.sync_copy(pos_r.at[wid], idx_p)
    pltpu.sync_copy(neg_r.at[wid], idx_n)

    iota16 = lax.iota(jnp.int32, 16)
    zero16 = jnp.full((16,), 0, jnp.int32)

    def chunk_body(ch, carry):
        def fire(g, carry2):
            gg = ch * CGROUP + g
            r = lax.shift_right_logical(gg, 2)
            sl = pl.ds(lax.bitwise_and(gg, 3) * 16, 16)
            uvec = idx_u[r, sl]
            pvec = idx_p[r, sl]
            nvec = idx_n[r, sl]
            base = g * 16
            for l in range(16):
                de = pl.ds(base + l, 1)
                pltpu.async_copy(uemb_r.at[pl.ds(uvec[l], 1), :],
                                 rows_u.at[de, :], sem)
                pltpu.async_copy(iemb_r.at[pl.ds(pvec[l], 1), :],
                                 rows_p.at[de, :], sem)
                pltpu.async_copy(iemb_r.at[pl.ds(nvec[l], 1), :],
                                 rows_n.at[de, :], sem)
                pltpu.async_copy(ibias_r.at[pl.ds(pvec[l], 1), :],
                                 bval_p.at[de, :], sem)
                pltpu.async_copy(ibias_r.at[pl.ds(nvec[l], 1), :],
                                 bval_n.at[de, :], sem)
            return carry2

        lax.fori_loop(0, CGROUP, fire, 0)

        pltpu.make_async_copy(uemb_r.at[pl.ds(0, CHUNK), :], rows_u, sem).wait()
        pltpu.make_async_copy(iemb_r.at[pl.ds(0, CHUNK), :], rows_p, sem).wait()
        pltpu.make_async_copy(iemb_r.at[pl.ds(0, CHUNK), :], rows_n, sem).wait()
        pltpu.make_async_copy(ibias_r.at[pl.ds(0, CHUNK), :], bval_p, sem).wait()
        pltpu.make_async_copy(ibias_r.at[pl.ds(0, CHUNK), :], bval_n, sem).wait()

        def group_body(g, carry2):
            gg = ch * CGROUP + g
            r = lax.shift_right_logical(gg, 2)
            sl = pl.ds(lax.bitwise_and(gg, 3) * 16, 16)
            e16 = g * 16 + iota16
            accp = plsc.load_gather(bval_p, [e16, zero16])
            accn = plsc.load_gather(bval_n, [e16, zero16])
            for d in range(D):
                dv = jnp.full((16,), d, jnp.int32)
                ul = plsc.load_gather(rows_u, [e16, dv])
                pv = plsc.load_gather(rows_p, [e16, dv])
                nv = plsc.load_gather(rows_n, [e16, dv])
                accp = accp + ul * pv
                accn = accn + ul * nv
            outp_v[r, sl] = accp
            outn_v[r, sl] = accn
            return carry2

        lax.fori_loop(0, CGROUP, group_body, 0)
        return carry

    lax.fori_loop(0, NCHUNK, chunk_body, 0)

    pltpu.sync_copy(outp_v, outp_hbm.at[wid])
    pltpu.sync_copy(outn_v, outn_hbm.at[wid])


_mesh = plsc.VectorSubcoreMesh(core_axis_name="c", subcore_axis_name="s",
                               num_cores=NUM_CORES, num_subcores=NUM_SUBCORES)

_sc_call = pl.kernel(
    _body,
    out_type=(jax.ShapeDtypeStruct((NW, NROW, NCOL), jnp.float32),
              jax.ShapeDtypeStruct((NW, NROW, NCOL), jnp.float32)),
    mesh=_mesh,
    scratch_types=[
        pltpu.VMEM((NROW, NCOL), jnp.int32),
        pltpu.VMEM((NROW, NCOL), jnp.int32),
        pltpu.VMEM((NROW, NCOL), jnp.int32),
        pltpu.VMEM((CHUNK, D), jnp.float32),
        pltpu.VMEM((CHUNK, D), jnp.float32),
        pltpu.VMEM((CHUNK, D), jnp.float32),
        pltpu.VMEM((CHUNK, 1), jnp.float32),
        pltpu.VMEM((CHUNK, 1), jnp.float32),
        pltpu.VMEM((NROW, NCOL), jnp.float32),
        pltpu.VMEM((NROW, NCOL), jnp.float32),
        pltpu.SemaphoreType.DMA,
    ],
    compiler_params=pltpu.CompilerParams(needs_layout_passes=False),
    name="bprmf_sc",
)


def kernel(users, pos_items, neg_items, user_emb, item_emb, item_bias):
    users3 = users.astype(jnp.int32).reshape(NW, NROW, NCOL)
    pos3 = pos_items.astype(jnp.int32).reshape(NW, NROW, NCOL)
    neg3 = neg_items.astype(jnp.int32).reshape(NW, NROW, NCOL)
    pos_s, neg_s = _sc_call(users3, pos3, neg3, user_emb, item_emb, item_bias)
    return pos_s.reshape(BATCH), neg_s.reshape(BATCH)
